# baseline (speedup 1.0000x reference)
import functools

import jax
import jax.numpy as jnp
from jax import lax
from jax.experimental import pallas as pl
from jax.experimental.pallas import tpu as pltpu
from jax.experimental.pallas import tpu_sc as plsc

_BATCH, _NT, _D = 4, 8192, 768
_NC, _NS = 2, 16
_NW = _NC * _NS
_TPW = _NT // _NW
_CHUNK = 16
_NCH = _TPW // _CHUNK


def _sc_body(tok_hbm, tab_hbm, out_hbm, *refs):
    toks = list(refs[0:8])
    tabs = list(refs[8:10])
    lsems = list(refs[10:18])
    ssems = [list(refs[18 + 2 * i:20 + 2 * i]) for i in range(8)]
    tsems = list(refs[34:36])

    wid = lax.axis_index("s") * _NC + lax.axis_index("c")
    t0 = wid * _TPW

    def tab_copy(ci, par):
        return pltpu.make_async_copy(
            tab_hbm.at[pl.ds(t0 + ci * _CHUNK, _CHUNK)], tabs[par], tsems[par]
        )

    def tok_copy(ci, b, slot):
        return pltpu.make_async_copy(
            tok_hbm.at[b, pl.ds(t0 + ci * _CHUNK, _CHUNK)],
            toks[slot], lsems[slot],
        )

    _H = _CHUNK // 2

    def out_copy(ci, b, slot, h):
        return pltpu.make_async_copy(
            toks[slot].at[pl.ds(h * _H, _H)],
            out_hbm.at[b, pl.ds(t0 + ci * _CHUNK + h * _H, _H)],
            ssems[slot][h],
        )

    tab_copy(0, 0).start()
    for b in range(_BATCH):
        tok_copy(0, b, b).start()

    def chunk_iter(ci, _):
        def body(par):
            cur = par * _BATCH
            nxt = (1 - par) * _BATCH
            tab_copy(ci, par).wait()

            @pl.when(ci + 1 < _NCH)
            def _():
                tab_copy(ci + 1, 1 - par).start()

            tab_v = tabs[par]
            for b in range(_BATCH):
                slot = cur + b

                @pl.when(ci > 0)
                def _():
                    out_copy(ci - 1, b, nxt + b, 0).wait()
                    out_copy(ci - 1, b, nxt + b, 1).wait()

                @pl.when(ci + 1 < _NCH)
                def _():
                    tok_copy(ci + 1, b, nxt + b).start()

                tok_copy(ci, b, slot).wait()
                tok_v = toks[slot]

                def add_row(r, _):
                    for c in range(_D // 16):
                        o = c * 16
                        plsc.addupdate(
                            tok_v.at[r, pl.ds(o, 16)], tab_v[r, pl.ds(o, 16)]
                        )
                    return 0

                lax.fori_loop(0, _H, add_row, 0)
                out_copy(ci, b, slot, 0).start()
                lax.fori_loop(_H, _CHUNK, add_row, 0)
                out_copy(ci, b, slot, 1).start()

        lax.cond(ci % 2 == 0, lambda: body(0), lambda: body(1))
        return 0

    lax.fori_loop(0, _NCH, chunk_iter, 0)

    for b in range(_BATCH):
        out_copy(_NCH - 1, b, _BATCH + b, 0).wait()
        out_copy(_NCH - 1, b, _BATCH + b, 1).wait()


def kernel(encoded_tokens, position_table):
    mesh = plsc.VectorSubcoreMesh(core_axis_name="c", subcore_axis_name="s")
    scratch = (
        [pltpu.VMEM((_CHUNK, _D), jnp.float32)] * 8
        + [pltpu.VMEM((_CHUNK, _D), jnp.float32)] * 2
        + [pltpu.SemaphoreType.DMA] * 26
    )
    run = functools.partial(
        pl.kernel,
        mesh=mesh,
        out_type=jax.ShapeDtypeStruct((_BATCH, _NT, _D), jnp.float32),
        scratch_types=scratch,
    )(_sc_body)
    return run(encoded_tokens, position_table)


# speedup vs baseline: 1.4392x; 1.4392x over previous
import functools

import jax
import jax.numpy as jnp
from jax import lax
from jax.experimental import pallas as pl
from jax.experimental.pallas import tpu as pltpu
from jax.experimental.pallas import tpu_sc as plsc

_BATCH, _NT, _D = 4, 8192, 768
_NC, _NS = 2, 16
_NW = _NC * _NS
_TPW = _NT // _NW
_CHUNK = 16
_NCH = _TPW // _CHUNK


def _sc_body(tok_hbm, tab_hbm, out_hbm, *refs):
    toks = list(refs[0:8])
    tabs = list(refs[8:10])
    lsems = list(refs[10:18])
    ssems = list(refs[18:26])
    tsems = list(refs[26:28])

    wid = lax.axis_index("s") * _NC + lax.axis_index("c")
    t0 = wid * _TPW

    def tab_copy(ci, par):
        return pltpu.make_async_copy(
            tab_hbm.at[pl.ds(t0 + ci * _CHUNK, _CHUNK)], tabs[par], tsems[par]
        )

    def tok_copy(ci, b, slot):
        return pltpu.make_async_copy(
            tok_hbm.at[b, pl.ds(t0 + ci * _CHUNK, _CHUNK)],
            toks[slot], lsems[slot],
        )

    def out_copy(ci, b, slot):
        return pltpu.make_async_copy(
            toks[slot], out_hbm.at[b, pl.ds(t0 + ci * _CHUNK, _CHUNK)],
            ssems[slot],
        )

    tab_copy(0, 0).start()
    for b in range(_BATCH):
        tok_copy(0, b, b).start()

    def chunk_iter(ci, _):
        def body(par):
            cur = par * _BATCH
            nxt = (1 - par) * _BATCH
            tab_copy(ci, par).wait()

            @pl.when(ci + 1 < _NCH)
            def _():
                tab_copy(ci + 1, 1 - par).start()

            tab_v = tabs[par]
            for b in range(_BATCH):
                slot = cur + b

                @pl.when(ci > 0)
                def _():
                    out_copy(ci - 1, b, nxt + b).wait()

                @pl.when(ci + 1 < _NCH)
                def _():
                    tok_copy(ci + 1, b, nxt + b).start()

                tok_copy(ci, b, slot).wait()
                tok_v = toks[slot]

                def add_row(r, _):
                    for c in range(_D // 16):
                        o = c * 16
                        plsc.addupdate(
                            tok_v.at[r, pl.ds(o, 16)], tab_v[r, pl.ds(o, 16)]
                        )
                    return 0

                lax.fori_loop(0, _CHUNK, add_row, 0)
                out_copy(ci, b, slot).start()

        lax.cond(ci % 2 == 0, lambda: body(0), lambda: body(1))
        return 0

    lax.fori_loop(0, _NCH, chunk_iter, 0)

    for b in range(_BATCH):
        out_copy(_NCH - 1, b, _BATCH + b).wait()


def kernel(encoded_tokens, position_table):
    mesh = plsc.VectorSubcoreMesh(core_axis_name="c", subcore_axis_name="s")
    scratch = (
        [pltpu.VMEM((_CHUNK, _D), jnp.float32)] * 8
        + [pltpu.VMEM((_CHUNK, _D), jnp.float32)] * 2
        + [pltpu.SemaphoreType.DMA] * 18
    )
    run = functools.partial(
        pl.kernel,
        mesh=mesh,
        out_type=jax.ShapeDtypeStruct((_BATCH, _NT, _D), jnp.float32),
        scratch_types=scratch,
    )(_sc_body)
    return run(encoded_tokens, position_table)
